# Initial kernel scaffold; baseline (speedup 1.0000x reference)
import jax
import jax.numpy as jnp
from jax.experimental import pallas as pl


def kernel(x, edge_index, W_e, b_e, W_d1, b_d1, W_d2, b_d2):
    raise NotImplementedError("write your pallas kernel here")


# speedup vs baseline: 5.4318x; 5.4318x over previous
import functools

import jax
import jax.numpy as jnp
from jax import lax
from jax.experimental import pallas as pl
from jax.experimental.pallas import tpu as pltpu
from jax.experimental.pallas import tpu_sc as plsc

N_TRUE = 10000
E_TRUE = 320000
D = 128
D_HID = 256
D_OUT = 64

NC = 2
NS = 16
NW = NC * NS

CHUNK = 128
N_PAD = 10240
ROWS_PER_SUB = N_PAD // NS
E_CHUNKS_PER_W = 79
EPT = E_CHUNKS_PER_W * CHUNK
E_PAD = EPT * NW

BN_TC = 1024
N_BLOCKS = N_PAD // BN_TC


def _sc_segment_sum(table, src_p, dst_p, zeros2d):
  mesh = plsc.VectorSubcoreMesh(core_axis_name="c", subcore_axis_name="s")

  @functools.partial(
      pl.kernel,
      mesh=mesh,
      out_type=[
          jax.ShapeDtypeStruct((NC, N_PAD, D), jnp.float32),
          jax.ShapeDtypeStruct((NW, N_PAD), jnp.float32),
      ],
      compiler_params=pltpu.CompilerParams(needs_layout_passes=False),
      scratch_types=[
          pltpu.VMEM((CHUNK,), jnp.int32),
          pltpu.VMEM((CHUNK,), jnp.int32),
          pltpu.VMEM((CHUNK, D), jnp.float32),
          pltpu.VMEM((N_PAD,), jnp.float32),
          pltpu.VMEM_SHARED((N_PAD, D), jnp.float32),
          pltpu.SemaphoreType.DMA,
      ],
  )
  def seg_kernel(tab_hbm, src_hbm, dst_hbm, z_hbm, agg_out, deg_out,
                 src_v, dst_v, rows_v, deg_v, agg_sh, sem):
    c = lax.axis_index("c")
    s = lax.axis_index("s")
    wid = c * NS + s

    pltpu.sync_copy(z_hbm.at[pl.ds(s * ROWS_PER_SUB, ROWS_PER_SUB)],
                    agg_sh.at[pl.ds(s * ROWS_PER_SUB, ROWS_PER_SUB)])

    zeros16 = jnp.zeros((16,), jnp.float32)
    def _zero_deg(i, carry):
      deg_v[pl.ds(i * 16, 16)] = zeros16
      return carry
    lax.fori_loop(0, N_PAD // 16, _zero_deg, 0)

    plsc.subcore_barrier()

    ones16 = jnp.full((16,), 1.0, jnp.float32)
    base = wid * EPT

    def _edge_chunk(k, carry):
      off = base + k * CHUNK
      pltpu.sync_copy(src_hbm.at[pl.ds(off, CHUNK)], src_v)
      pltpu.sync_copy(dst_hbm.at[pl.ds(off, CHUNK)], dst_v)
      pltpu.async_copy(tab_hbm.at[src_v], rows_v, sem).wait()
      pltpu.sync_copy(rows_v, agg_sh.at[dst_v], add=True)
      for j in range(CHUNK // 16):
        idx = dst_v[pl.ds(j * 16, 16)]
        plsc.addupdate_scatter(deg_v, [idx], ones16)
      return carry

    lax.fori_loop(0, E_CHUNKS_PER_W, _edge_chunk, 0)

    plsc.subcore_barrier()

    pltpu.sync_copy(agg_sh.at[pl.ds(s * ROWS_PER_SUB, ROWS_PER_SUB)],
                    agg_out.at[c, pl.ds(s * ROWS_PER_SUB, ROWS_PER_SUB)])
    pltpu.sync_copy(deg_v, deg_out.at[wid])

  return seg_kernel(table, src_p, dst_p, zeros2d)


def _tc_decode_body(x_ref, agg_ref, deg_ref, we_ref, be_ref, wd1_ref, bd1_ref,
                    wd2_ref, bd2_ref, out_ref, u_acc):
  i = pl.program_id(0)

  @pl.when(i == 0)
  def _():
    u_acc[...] = jnp.zeros((8, D), jnp.float32)

  agg = agg_ref[0] + agg_ref[1]
  deg = jnp.sum(deg_ref[...], axis=0)[:, None]
  mean = agg / jnp.maximum(deg, 1.0)
  z = (x_ref[...] + mean) @ we_ref[...] + be_ref[...]
  h = jnp.maximum(z, 0.0)

  row = i * BN_TC + lax.broadcasted_iota(jnp.int32, (BN_TC, 1), 0)
  h = jnp.where(row < N_TRUE, h, 0.0)
  u_acc[0:1, :] += jnp.sum(h, axis=0, keepdims=True)

  @pl.when(i == N_BLOCKS - 1)
  def _():
    u = u_acc[0:1, :] * (1.0 / N_TRUE)
    hid = jnp.maximum(u @ wd1_ref[...] + bd1_ref[...], 0.0)
    out_ref[...] = hid @ wd2_ref[...] + bd2_ref[...]


def _tc_decode(x_pad, agg2, deg32, W_e, b_e, W_d1, b_d1, W_d2, b_d2):
  out = pl.pallas_call(
      _tc_decode_body,
      grid=(N_BLOCKS,),
      in_specs=[
          pl.BlockSpec((BN_TC, D), lambda i: (i, 0)),
          pl.BlockSpec((NC, BN_TC, D), lambda i: (0, i, 0)),
          pl.BlockSpec((NW, BN_TC), lambda i: (0, i)),
          pl.BlockSpec((D, D), lambda i: (0, 0)),
          pl.BlockSpec((1, D), lambda i: (0, 0)),
          pl.BlockSpec((D, D_HID), lambda i: (0, 0)),
          pl.BlockSpec((1, D_HID), lambda i: (0, 0)),
          pl.BlockSpec((D_HID, D_OUT), lambda i: (0, 0)),
          pl.BlockSpec((1, D_OUT), lambda i: (0, 0)),
      ],
      out_specs=pl.BlockSpec((1, D_OUT), lambda i: (0, 0)),
      out_shape=jax.ShapeDtypeStruct((1, D_OUT), jnp.float32),
      scratch_shapes=[pltpu.VMEM((8, D), jnp.float32)],
  )(x_pad, agg2, deg32, W_e, b_e.reshape(1, D), W_d1, b_d1.reshape(1, D_HID),
    W_d2, b_d2.reshape(1, D_OUT))
  return out.reshape(D_OUT)


@jax.jit
def kernel(x, edge_index, W_e, b_e, W_d1, b_d1, W_d2, b_d2):
  src = edge_index[0].astype(jnp.int32)
  dst = edge_index[1].astype(jnp.int32)
  pad_idx = jnp.full((E_PAD - E_TRUE,), N_TRUE, jnp.int32)
  src_p = jnp.concatenate([src, pad_idx])
  dst_p = jnp.concatenate([dst, pad_idx])
  x_pad = jnp.pad(x, ((0, N_PAD - N_TRUE), (0, 0)))
  zeros2d = jnp.zeros((N_PAD, D), jnp.float32)

  agg2, deg32 = _sc_segment_sum(x_pad, src_p, dst_p, zeros2d)
  return _tc_decode(x_pad, agg2, deg32, W_e, b_e, W_d1, b_d1, W_d2, b_d2)
